# Initial kernel scaffold; baseline (speedup 1.0000x reference)
import functools

import jax
import jax.numpy as jnp
from jax import lax
from jax.experimental import pallas as pl
from jax.experimental.pallas import tpu as pltpu
from jax.experimental.pallas import tpu_sc as plsc

N_NODES = 50000
N_EDGES = 800000
D_IN = 128
D_H = 64

NC = 2
NS = 16
HALF = N_NODES // NC
STRIPE = 1563
ROWS = NS * STRIPE
TRASH = HALF
EPS = N_EDGES // NS
G = 128
NCH = EPS // G

_sc_mesh = plsc.VectorSubcoreMesh(core_axis_name="c", subcore_axis_name="s")


def _localize(dst_v, base):
    for k in range(G // 16):
        d = dst_v[pl.ds(k * 16, 16)]
        m = (d >= base) & (d < base + HALF)
        dst_v[pl.ds(k * 16, 16)] = jnp.where(m, d - base, TRASH)


@functools.partial(
    pl.kernel,
    out_type=jax.ShapeDtypeStruct((NC, ROWS, D_H), jnp.float32),
    mesh=_sc_mesh,
    scratch_types=[
        pltpu.VMEM((G,), jnp.int32),
        pltpu.VMEM((G,), jnp.int32),
        pltpu.VMEM((G, D_H), jnp.float32),
        pltpu.VMEM_SHARED((ROWS, D_H), jnp.float32),
        pltpu.SemaphoreType.DMA,
    ],
)
def _sc_agg(h_hbm, src_hbm, dst_hbm, zeros_hbm, out_hbm, src_v, ldst_v, rows_v, acc_sh, sem):
    c = lax.axis_index("c")
    s = lax.axis_index("s")
    base = c * HALF
    pltpu.sync_copy(zeros_hbm, acc_sh.at[pl.ds(s * STRIPE, STRIPE)])
    plsc.subcore_barrier()

    def chunk(j, carry):
        off = s * EPS + j * G
        pltpu.sync_copy(dst_hbm.at[pl.ds(off, G)], ldst_v)
        pltpu.sync_copy(src_hbm.at[pl.ds(off, G)], src_v)
        _localize(ldst_v, base)
        pltpu.async_copy(h_hbm.at[src_v], rows_v, sem).wait()
        pltpu.sync_copy(rows_v, acc_sh.at[ldst_v], add=True)
        return carry

    lax.fori_loop(0, NCH, chunk, 0)
    plsc.subcore_barrier()
    pltpu.sync_copy(acc_sh.at[pl.ds(s * STRIPE, STRIPE)],
                    out_hbm.at[c].at[pl.ds(s * STRIPE, STRIPE)])


@functools.partial(
    pl.kernel,
    out_type=jax.ShapeDtypeStruct((NC, ROWS, 16), jnp.float32),
    mesh=_sc_mesh,
    scratch_types=[
        pltpu.VMEM((G,), jnp.int32),
        pltpu.VMEM((G, 16), jnp.float32),
        pltpu.VMEM_SHARED((ROWS, 16), jnp.float32),
    ],
)
def _sc_counts(dst_hbm, zeros_hbm, ones_hbm, out_hbm, ldst_v, ones_v, cnt_sh):
    c = lax.axis_index("c")
    s = lax.axis_index("s")
    base = c * HALF
    pltpu.sync_copy(zeros_hbm, cnt_sh.at[pl.ds(s * STRIPE, STRIPE)])
    pltpu.sync_copy(ones_hbm, ones_v)
    plsc.subcore_barrier()

    def chunk(j, carry):
        off = s * EPS + j * G
        pltpu.sync_copy(dst_hbm.at[pl.ds(off, G)], ldst_v)
        _localize(ldst_v, base)
        pltpu.sync_copy(ones_v, cnt_sh.at[ldst_v], add=True)
        return carry

    lax.fori_loop(0, NCH, chunk, 0)
    plsc.subcore_barrier()
    pltpu.sync_copy(cnt_sh.at[pl.ds(s * STRIPE, STRIPE)],
                    out_hbm.at[c].at[pl.ds(s * STRIPE, STRIPE)])


_R = 2000


def _mlp_in_body(x_ref, w_ref, b_ref, o_ref):
    o_ref[...] = jnp.maximum(
        jnp.dot(x_ref[...], w_ref[...], preferred_element_type=jnp.float32)
        + b_ref[...], 0.0)


def _mlp_in(x, w, b):
    return pl.pallas_call(
        _mlp_in_body,
        grid=(N_NODES // _R,),
        in_specs=[
            pl.BlockSpec((_R, D_IN), lambda i: (i, 0)),
            pl.BlockSpec((D_IN, D_H), lambda i: (0, 0)),
            pl.BlockSpec((1, D_H), lambda i: (0, 0)),
        ],
        out_specs=pl.BlockSpec((_R, D_H), lambda i: (i, 0)),
        out_shape=jax.ShapeDtypeStruct((N_NODES, D_H), jnp.float32),
    )(x, w, b)


def _combine_body(h_ref, agg_ref, cnt_ref, ws_ref, bs_ref, wn_ref, bn_ref,
                  g_ref, be_ref, o_ref, *, last):
    h = h_ref[...]
    self_f = jnp.dot(h, ws_ref[...], preferred_element_type=jnp.float32) + bs_ref[...]
    cnt = jnp.maximum(cnt_ref[...][:, 0:1], 1.0)
    agg = agg_ref[...] / cnt
    neigh = jnp.dot(agg, wn_ref[...], preferred_element_type=jnp.float32) + bn_ref[...]
    t = self_f + neigh
    mu = jnp.mean(t, axis=-1, keepdims=True)
    var = jnp.mean((t - mu) ** 2, axis=-1, keepdims=True)
    t = (t - mu) / jnp.sqrt(var + 1e-5) * g_ref[...] + be_ref[...]
    if not last:
        t = jnp.maximum(t, 0.0) + h
    o_ref[...] = t


def _combine(h, agg, cnt, ws, bs, wn, bn, g, be, last):
    return pl.pallas_call(
        functools.partial(_combine_body, last=last),
        grid=(N_NODES // _R,),
        in_specs=[
            pl.BlockSpec((_R, D_H), lambda i: (i, 0)),
            pl.BlockSpec((_R, D_H), lambda i: (i, 0)),
            pl.BlockSpec((_R, 16), lambda i: (i, 0)),
            pl.BlockSpec((D_H, D_H), lambda i: (0, 0)),
            pl.BlockSpec((1, D_H), lambda i: (0, 0)),
            pl.BlockSpec((D_H, D_H), lambda i: (0, 0)),
            pl.BlockSpec((1, D_H), lambda i: (0, 0)),
            pl.BlockSpec((1, D_H), lambda i: (0, 0)),
            pl.BlockSpec((1, D_H), lambda i: (0, 0)),
        ],
        out_specs=pl.BlockSpec((_R, D_H), lambda i: (i, 0)),
        out_shape=jax.ShapeDtypeStruct((N_NODES, D_H), jnp.float32),
    )(h, agg, cnt, ws, bs, wn, bn, g, be)


def _merge_halves(y):
    return jnp.concatenate([y[0, :HALF], y[1, :HALF]], axis=0)


def kernel(x, edge_index, W_in, b_in, Ws0, bs0, Wn0, bn0, g0, be0,
           Ws1, bs1, Wn1, bn1, g1, be1):
    src = edge_index[0].astype(jnp.int32)
    dst = edge_index[1].astype(jnp.int32)
    zeros64 = jnp.zeros((STRIPE, D_H), jnp.float32)
    zeros16 = jnp.zeros((STRIPE, 16), jnp.float32)
    ones16 = jnp.ones((G, 16), jnp.float32)

    h0 = _mlp_in(x, W_in, b_in.reshape(1, -1))
    cnt = _merge_halves(_sc_counts(dst, zeros16, ones16))

    agg0 = _merge_halves(_sc_agg(h0, src, dst, zeros64))
    h1 = _combine(h0, agg0, cnt, Ws0, bs0.reshape(1, -1), Wn0, bn0.reshape(1, -1),
                  g0.reshape(1, -1), be0.reshape(1, -1), last=False)

    agg1 = _merge_halves(_sc_agg(h1, src, dst, zeros64))
    out = _combine(h1, agg1, cnt, Ws1, bs1.reshape(1, -1), Wn1, bn1.reshape(1, -1),
                   g1.reshape(1, -1), be1.reshape(1, -1), last=True)
    return out


# speedup vs baseline: 3.0811x; 3.0811x over previous
import functools

import jax
import jax.numpy as jnp
from jax import lax
from jax.experimental import pallas as pl
from jax.experimental.pallas import tpu as pltpu
from jax.experimental.pallas import tpu_sc as plsc

N_NODES = 50000
N_EDGES = 800000
D_IN = 128
D_H = 64

NC = 2
NS = 16
HALF = N_NODES // NC
STRIPE = 1568
ROWS = NS * STRIPE
TRASH = HALF
EPS = N_EDGES // NS
G = 128
NCH = EPS // G

_sc_mesh = plsc.VectorSubcoreMesh(core_axis_name="c", subcore_axis_name="s")


def _localize(dst_v, base):
    for k in range(G // 16):
        d = dst_v[pl.ds(k * 16, 16)]
        m = (d >= base) & (d < base + HALF)
        dst_v[pl.ds(k * 16, 16)] = jnp.where(m, d - base, TRASH)


@functools.partial(
    pl.kernel,
    out_type=jax.ShapeDtypeStruct((NC, ROWS, D_H), jnp.float32),
    mesh=_sc_mesh,
    scratch_types=[
        pltpu.VMEM((G,), jnp.int32),
        pltpu.VMEM((G,), jnp.int32),
        pltpu.VMEM((G, D_H), jnp.float32),
        pltpu.VMEM_SHARED((ROWS, D_H), jnp.float32),
        pltpu.SemaphoreType.DMA,
    ],
    compiler_params=pltpu.CompilerParams(use_tc_tiling_on_sc=False),
)
def _sc_agg(h_hbm, src_hbm, dst_hbm, zeros_hbm, out_hbm, src_v, ldst_v, rows_v, acc_sh, sem):
    c = lax.axis_index("c")
    s = lax.axis_index("s")
    base = c * HALF
    pltpu.sync_copy(zeros_hbm, acc_sh.at[pl.ds(s * STRIPE, STRIPE)])
    plsc.subcore_barrier()

    def chunk(j, carry):
        off = s * EPS + j * G
        pltpu.sync_copy(dst_hbm.at[pl.ds(off, G)], ldst_v)
        pltpu.sync_copy(src_hbm.at[pl.ds(off, G)], src_v)
        _localize(ldst_v, base)
        pltpu.async_copy(h_hbm.at[src_v], rows_v, sem).wait()
        pltpu.sync_copy(rows_v, acc_sh.at[ldst_v], add=True)
        return carry

    lax.fori_loop(0, NCH, chunk, 0)
    plsc.subcore_barrier()
    pltpu.sync_copy(acc_sh.at[pl.ds(s * STRIPE, STRIPE)],
                    out_hbm.at[c].at[pl.ds(s * STRIPE, STRIPE)])


@functools.partial(
    pl.kernel,
    out_type=jax.ShapeDtypeStruct((NC, ROWS, 16), jnp.float32),
    mesh=_sc_mesh,
    scratch_types=[
        pltpu.VMEM((G,), jnp.int32),
        pltpu.VMEM((G, 16), jnp.float32),
        pltpu.VMEM_SHARED((ROWS, 16), jnp.float32),
    ],
    compiler_params=pltpu.CompilerParams(use_tc_tiling_on_sc=False),
)
def _sc_counts(dst_hbm, zeros_hbm, ones_hbm, out_hbm, ldst_v, ones_v, cnt_sh):
    c = lax.axis_index("c")
    s = lax.axis_index("s")
    base = c * HALF
    pltpu.sync_copy(zeros_hbm, cnt_sh.at[pl.ds(s * STRIPE, STRIPE)])
    pltpu.sync_copy(ones_hbm, ones_v)
    plsc.subcore_barrier()

    def chunk(j, carry):
        off = s * EPS + j * G
        pltpu.sync_copy(dst_hbm.at[pl.ds(off, G)], ldst_v)
        _localize(ldst_v, base)
        pltpu.sync_copy(ones_v, cnt_sh.at[ldst_v], add=True)
        return carry

    lax.fori_loop(0, NCH, chunk, 0)
    plsc.subcore_barrier()
    pltpu.sync_copy(cnt_sh.at[pl.ds(s * STRIPE, STRIPE)],
                    out_hbm.at[c].at[pl.ds(s * STRIPE, STRIPE)])


_R = 2000


def _mlp_in_body(x_ref, w_ref, b_ref, o_ref):
    o_ref[...] = jnp.maximum(
        jnp.dot(x_ref[...], w_ref[...], preferred_element_type=jnp.float32)
        + b_ref[...], 0.0)


def _mlp_in(x, w, b):
    return pl.pallas_call(
        _mlp_in_body,
        grid=(N_NODES // _R,),
        in_specs=[
            pl.BlockSpec((_R, D_IN), lambda i: (i, 0)),
            pl.BlockSpec((D_IN, D_H), lambda i: (0, 0)),
            pl.BlockSpec((1, D_H), lambda i: (0, 0)),
        ],
        out_specs=pl.BlockSpec((_R, D_H), lambda i: (i, 0)),
        out_shape=jax.ShapeDtypeStruct((N_NODES, D_H), jnp.float32),
    )(x, w, b)


def _combine_body(h_ref, agg_ref, cnt_ref, ws_ref, bs_ref, wn_ref, bn_ref,
                  g_ref, be_ref, o_ref, *, last):
    h = h_ref[...]
    self_f = jnp.dot(h, ws_ref[...], preferred_element_type=jnp.float32) + bs_ref[...]
    cnt = jnp.maximum(cnt_ref[...][:, 0:1], 1.0)
    agg = agg_ref[...] / cnt
    neigh = jnp.dot(agg, wn_ref[...], preferred_element_type=jnp.float32) + bn_ref[...]
    t = self_f + neigh
    mu = jnp.mean(t, axis=-1, keepdims=True)
    var = jnp.mean((t - mu) ** 2, axis=-1, keepdims=True)
    t = (t - mu) / jnp.sqrt(var + 1e-5) * g_ref[...] + be_ref[...]
    if not last:
        t = jnp.maximum(t, 0.0) + h
    o_ref[...] = t


def _combine(h, agg, cnt, ws, bs, wn, bn, g, be, last):
    return pl.pallas_call(
        functools.partial(_combine_body, last=last),
        grid=(N_NODES // _R,),
        in_specs=[
            pl.BlockSpec((_R, D_H), lambda i: (i, 0)),
            pl.BlockSpec((_R, D_H), lambda i: (i, 0)),
            pl.BlockSpec((_R, 16), lambda i: (i, 0)),
            pl.BlockSpec((D_H, D_H), lambda i: (0, 0)),
            pl.BlockSpec((1, D_H), lambda i: (0, 0)),
            pl.BlockSpec((D_H, D_H), lambda i: (0, 0)),
            pl.BlockSpec((1, D_H), lambda i: (0, 0)),
            pl.BlockSpec((1, D_H), lambda i: (0, 0)),
            pl.BlockSpec((1, D_H), lambda i: (0, 0)),
        ],
        out_specs=pl.BlockSpec((_R, D_H), lambda i: (i, 0)),
        out_shape=jax.ShapeDtypeStruct((N_NODES, D_H), jnp.float32),
    )(h, agg, cnt, ws, bs, wn, bn, g, be)


def _merge_halves(y):
    return jnp.concatenate([y[0, :HALF], y[1, :HALF]], axis=0)


def kernel(x, edge_index, W_in, b_in, Ws0, bs0, Wn0, bn0, g0, be0,
           Ws1, bs1, Wn1, bn1, g1, be1):
    src = edge_index[0].astype(jnp.int32)
    dst = edge_index[1].astype(jnp.int32)
    zeros64 = jnp.zeros((STRIPE, D_H), jnp.float32)
    zeros16 = jnp.zeros((STRIPE, 16), jnp.float32)
    ones16 = jnp.ones((G, 16), jnp.float32)

    h0 = _mlp_in(x, W_in, b_in.reshape(1, -1))
    cnt = _merge_halves(_sc_counts(dst, zeros16, ones16))

    agg0 = _merge_halves(_sc_agg(h0, src, dst, zeros64))
    h1 = _combine(h0, agg0, cnt, Ws0, bs0.reshape(1, -1), Wn0, bn0.reshape(1, -1),
                  g0.reshape(1, -1), be0.reshape(1, -1), last=False)

    agg1 = _merge_halves(_sc_agg(h1, src, dst, zeros64))
    out = _combine(h1, agg1, cnt, Ws1, bs1.reshape(1, -1), Wn1, bn1.reshape(1, -1),
                   g1.reshape(1, -1), be1.reshape(1, -1), last=True)
    return out


# speedup vs baseline: 4.2621x; 1.3833x over previous
import functools

import jax
import jax.numpy as jnp
from jax import lax
from jax.experimental import pallas as pl
from jax.experimental.pallas import tpu as pltpu
from jax.experimental.pallas import tpu_sc as plsc

N_NODES = 50000
N_EDGES = 800000
D_IN = 128
D_H = 64

NC = 2
NS = 16
HALF = N_NODES // NC
STRIPE = 1568
ROWS = NS * STRIPE
TRASH = HALF
G = 80
RT = N_EDGES // G
NCH = RT // NS
K = 5

_sc_mesh = plsc.VectorSubcoreMesh(core_axis_name="c", subcore_axis_name="s")
_sc_params = pltpu.CompilerParams(use_tc_tiling_on_sc=False)


def _localize(base, ldst_v, slot):
    for q in range(G // 16):
        d = ldst_v[slot, pl.ds(q * 16, 16)]
        m = (d >= base) & (d < base + HALF)
        ldst_v[slot, pl.ds(q * 16, 16)] = jnp.where(m, d - base, TRASH)


def _make_agg():
    scratch = [
        pltpu.VMEM((K, G), jnp.int32),
        pltpu.VMEM((K, G), jnp.int32),
        pltpu.VMEM((K, G, D_H), jnp.float32),
        pltpu.VMEM_SHARED((ROWS, D_H), jnp.float32),
        pltpu.SemaphoreType.DMA,
        pltpu.SemaphoreType.DMA,
    ]

    def body(h_hbm, src2, dst2, z64, agg_out,
             src_v, ldst_v, rows_v, acc_sh, sem_i, sem_g):
        c = lax.axis_index("c")
        s = lax.axis_index("s")
        base = c * HALF
        start = s * NCH

        def fire_idx(j):
            slot = lax.rem(j, K)
            pltpu.async_copy(src2.at[start + j], src_v.at[slot], sem_i)
            pltpu.async_copy(dst2.at[start + j], ldst_v.at[slot], sem_i)

        def drain_fire_gather(j):
            slot = lax.rem(j, K)
            pltpu.make_async_copy(src2.at[0], src_v.at[slot], sem_i).wait()
            pltpu.make_async_copy(dst2.at[0], ldst_v.at[slot], sem_i).wait()
            pltpu.async_copy(h_hbm.at[src_v.at[slot]], rows_v.at[slot], sem_g)

        def prol_fire(j, carry):
            fire_idx(j)
            return carry

        lax.fori_loop(0, K, prol_fire, 0)
        pltpu.sync_copy(z64, acc_sh.at[pl.ds(s * STRIPE, STRIPE)])
        plsc.subcore_barrier()

        def prol_gather(j, carry):
            drain_fire_gather(j)
            return carry

        lax.fori_loop(0, K - 1, prol_gather, 0)

        def chunk_body(j, carry):
            slot = lax.rem(j, K)
            pltpu.make_async_copy(h_hbm.at[src_v.at[slot]], rows_v.at[slot],
                                  sem_g).wait()
            _localize(base, ldst_v, slot)
            pltpu.sync_copy(rows_v.at[slot], acc_sh.at[ldst_v.at[slot]],
                            add=True)

            @pl.when(j + K - 1 < NCH)
            def _():
                drain_fire_gather(j + K - 1)

            @pl.when(j + K < NCH)
            def _():
                fire_idx(j + K)

            return carry

        lax.fori_loop(0, NCH, chunk_body, 0)
        plsc.subcore_barrier()
        pltpu.sync_copy(acc_sh.at[pl.ds(s * STRIPE, STRIPE)],
                        agg_out.at[c].at[pl.ds(s * STRIPE, STRIPE)])

    return pl.kernel(
        body,
        out_type=jax.ShapeDtypeStruct((NC, ROWS, D_H), jnp.float32),
        mesh=_sc_mesh,
        scratch_types=scratch,
        compiler_params=_sc_params,
    )


def _make_counts():
    scratch = [
        pltpu.VMEM((2, G), jnp.int32),
        pltpu.VMEM((G, 16), jnp.float32),
        pltpu.VMEM_SHARED((ROWS, 16), jnp.float32),
        pltpu.SemaphoreType.DMA,
    ]

    def body(dst2, z16, ones_hbm, cnt_out, ldst_v, ones_v, cnt_sh, sem_i):
        c = lax.axis_index("c")
        s = lax.axis_index("s")
        base = c * HALF
        start = s * NCH

        pltpu.async_copy(dst2.at[start], ldst_v.at[0], sem_i)
        pltpu.sync_copy(z16, cnt_sh.at[pl.ds(s * STRIPE, STRIPE)])
        pltpu.sync_copy(ones_hbm, ones_v)
        plsc.subcore_barrier()

        def chunk_body(j, carry):
            slot = lax.rem(j, 2)
            pltpu.make_async_copy(dst2.at[0], ldst_v.at[slot], sem_i).wait()

            @pl.when(j + 1 < NCH)
            def _():
                pltpu.async_copy(dst2.at[start + j + 1], ldst_v.at[1 - slot],
                                 sem_i)

            _localize(base, ldst_v, slot)
            pltpu.sync_copy(ones_v, cnt_sh.at[ldst_v.at[slot]], add=True)
            return carry

        lax.fori_loop(0, NCH, chunk_body, 0)
        plsc.subcore_barrier()
        pltpu.sync_copy(cnt_sh.at[pl.ds(s * STRIPE, STRIPE)],
                        cnt_out.at[c].at[pl.ds(s * STRIPE, STRIPE)])

    return pl.kernel(
        body,
        out_type=jax.ShapeDtypeStruct((NC, ROWS, 16), jnp.float32),
        mesh=_sc_mesh,
        scratch_types=scratch,
        compiler_params=_sc_params,
    )


_sc_agg = _make_agg()
_sc_counts = _make_counts()


_R = 2000
_PREC = lax.Precision.HIGHEST


def _mlp_in_body(x_ref, w_ref, b_ref, o_ref):
    o_ref[...] = jnp.maximum(
        jnp.dot(x_ref[...], w_ref[...], preferred_element_type=jnp.float32,
                precision=_PREC) + b_ref[...], 0.0)


def _mlp_in(x, w, b):
    return pl.pallas_call(
        _mlp_in_body,
        grid=(N_NODES // _R,),
        in_specs=[
            pl.BlockSpec((_R, D_IN), lambda i: (i, 0)),
            pl.BlockSpec((D_IN, D_H), lambda i: (0, 0)),
            pl.BlockSpec((1, D_H), lambda i: (0, 0)),
        ],
        out_specs=pl.BlockSpec((_R, D_H), lambda i: (i, 0)),
        out_shape=jax.ShapeDtypeStruct((N_NODES, D_H), jnp.float32),
    )(x, w, b)


def _combine_body(h_ref, agg_ref, cnt_ref, ws_ref, bs_ref, wn_ref, bn_ref,
                  g_ref, be_ref, o_ref, *, last):
    h = h_ref[...]
    self_f = jnp.dot(h, ws_ref[...], preferred_element_type=jnp.float32,
                     precision=_PREC) + bs_ref[...]
    cnt = jnp.maximum(cnt_ref[...][:, 0:1], 1.0)
    agg = agg_ref[...] / cnt
    neigh = jnp.dot(agg, wn_ref[...], preferred_element_type=jnp.float32,
                    precision=_PREC) + bn_ref[...]
    t = self_f + neigh
    mu = jnp.mean(t, axis=-1, keepdims=True)
    var = jnp.mean((t - mu) ** 2, axis=-1, keepdims=True)
    t = (t - mu) / jnp.sqrt(var + 1e-5) * g_ref[...] + be_ref[...]
    if not last:
        t = jnp.maximum(t, 0.0) + h
    o_ref[...] = t


def _combine(h, agg, cnt, ws, bs, wn, bn, g, be, last):
    return pl.pallas_call(
        functools.partial(_combine_body, last=last),
        grid=(N_NODES // _R,),
        in_specs=[
            pl.BlockSpec((_R, D_H), lambda i: (i, 0)),
            pl.BlockSpec((_R, D_H), lambda i: (i, 0)),
            pl.BlockSpec((_R, 16), lambda i: (i, 0)),
            pl.BlockSpec((D_H, D_H), lambda i: (0, 0)),
            pl.BlockSpec((1, D_H), lambda i: (0, 0)),
            pl.BlockSpec((D_H, D_H), lambda i: (0, 0)),
            pl.BlockSpec((1, D_H), lambda i: (0, 0)),
            pl.BlockSpec((1, D_H), lambda i: (0, 0)),
            pl.BlockSpec((1, D_H), lambda i: (0, 0)),
        ],
        out_specs=pl.BlockSpec((_R, D_H), lambda i: (i, 0)),
        out_shape=jax.ShapeDtypeStruct((N_NODES, D_H), jnp.float32),
    )(h, agg, cnt, ws, bs, wn, bn, g, be)


def _merge_halves(y):
    return jnp.concatenate([y[0, :HALF], y[1, :HALF]], axis=0)


def kernel(x, edge_index, W_in, b_in, Ws0, bs0, Wn0, bn0, g0, be0,
           Ws1, bs1, Wn1, bn1, g1, be1):
    src2 = edge_index[0].astype(jnp.int32).reshape(RT, G)
    dst2 = edge_index[1].astype(jnp.int32).reshape(RT, G)
    zeros64 = jnp.zeros((STRIPE, D_H), jnp.float32)
    zeros16 = jnp.zeros((STRIPE, 16), jnp.float32)
    ones16 = jnp.ones((G, 16), jnp.float32)

    h0 = _mlp_in(x, W_in, b_in.reshape(1, -1))

    cnt = _merge_halves(_sc_counts(dst2, zeros16, ones16))
    agg0 = _merge_halves(_sc_agg(h0, src2, dst2, zeros64))
    h1 = _combine(h0, agg0, cnt, Ws0, bs0.reshape(1, -1), Wn0, bn0.reshape(1, -1),
                  g0.reshape(1, -1), be0.reshape(1, -1), last=False)

    agg1 = _merge_halves(_sc_agg(h1, src2, dst2, zeros64))
    out = _combine(h1, agg1, cnt, Ws1, bs1.reshape(1, -1), Wn1, bn1.reshape(1, -1),
                   g1.reshape(1, -1), be1.reshape(1, -1), last=True)
    return out
